# baseline (speedup 1.0000x reference)
import functools

import jax
import jax.numpy as jnp
from jax.experimental import pallas as pl
from jax.experimental.pallas import tpu as pltpu


def _se_block_kernel(x_ref, w1t_ref, b1t_ref, w2t_ref, b2t_ref, o_ref,
                     s_ref, *, nb, inv_hw):
    k = pl.program_id(1)
    half = nb // 2

    @pl.when(k == 0)
    def _():
        y = jnp.concatenate(
            [jnp.sum(x_ref[i], axis=1, keepdims=True) for i in range(nb)],
            axis=1,
        ) * inv_hw
        h = jax.lax.dot_general(
            w1t_ref[...], y,
            dimension_numbers=(((1,), (0,)), ((), ())),
            preferred_element_type=jnp.float32,
        ) + b1t_ref[...]
        h = jnp.maximum(h, 0.0)
        z = jax.lax.dot_general(
            w2t_ref[...], h,
            dimension_numbers=(((1,), (0,)), ((), ())),
            preferred_element_type=jnp.float32,
        ) + b2t_ref[...]
        s_ref[...] = jax.nn.sigmoid(z)

    @pl.when(k == 0)
    def _():
        for j in range(half):
            o_ref[j] = x_ref[j] * s_ref[:, j:j + 1]

    @pl.when(k == 1)
    def _():
        for j in range(half):
            o_ref[j] = x_ref[half + j] * s_ref[:, half + j:half + j + 1]


def _se_single_kernel(x_ref, w1t_ref, b1t_ref, w2t_ref, b2t_ref, o_ref, *,
                      inv_hw):
    x = x_ref[0]
    y = jnp.sum(x, axis=1, keepdims=True) * inv_hw
    h = jax.lax.dot_general(
        w1t_ref[...], y,
        dimension_numbers=(((1,), (0,)), ((), ())),
        preferred_element_type=jnp.float32,
    ) + b1t_ref[...]
    h = jnp.maximum(h, 0.0)
    z = jax.lax.dot_general(
        w2t_ref[...], h,
        dimension_numbers=(((1,), (0,)), ((), ())),
        preferred_element_type=jnp.float32,
    ) + b2t_ref[...]
    s = jax.nn.sigmoid(z)
    o_ref[0] = x * s


def kernel(x, w1, b1, w2, b2):
    B, C, H, W = x.shape
    HW = H * W
    hidden = w1.shape[1]
    x3d = x.reshape(B, C, HW)

    slab_bytes = C * HW * x.dtype.itemsize
    nb = 1
    for cand in (4, 2):
        if B % cand == 0 and 4 * cand * slab_bytes <= 52 * 1024 * 1024:
            nb = cand
            break

    w1t = w1.T
    b1t = b1.reshape(hidden, 1)
    w2t = w2.T
    b2t = b2.reshape(C, 1)

    if nb >= 2:
        half = nb // 2
        out3d = pl.pallas_call(
            functools.partial(_se_block_kernel, nb=nb, inv_hw=1.0 / HW),
            out_shape=jax.ShapeDtypeStruct((B, C, HW), x3d.dtype),
            grid=(B // nb, 2),
            in_specs=[
                pl.BlockSpec((nb, C, HW), lambda b, k: (b, 0, 0)),
                pl.BlockSpec((hidden, C), lambda b, k: (0, 0)),
                pl.BlockSpec((hidden, 1), lambda b, k: (0, 0)),
                pl.BlockSpec((C, hidden), lambda b, k: (0, 0)),
                pl.BlockSpec((C, 1), lambda b, k: (0, 0)),
            ],
            out_specs=pl.BlockSpec((half, C, HW),
                                   lambda b, k: (2 * b + k, 0, 0)),
            scratch_shapes=[pltpu.VMEM((C, nb), jnp.float32)],
            compiler_params=pltpu.CompilerParams(
                dimension_semantics=("parallel", "arbitrary"),
                vmem_limit_bytes=56 * 1024 * 1024,
            ),
        )(x3d, w1t, b1t, w2t, b2t)
    else:
        out3d = pl.pallas_call(
            functools.partial(_se_single_kernel, inv_hw=1.0 / HW),
            out_shape=jax.ShapeDtypeStruct((B, C, HW), x3d.dtype),
            grid=(B,),
            in_specs=[
                pl.BlockSpec((1, C, HW), lambda b: (b, 0, 0)),
                pl.BlockSpec((hidden, C), lambda b: (0, 0)),
                pl.BlockSpec((hidden, 1), lambda b: (0, 0)),
                pl.BlockSpec((C, hidden), lambda b: (0, 0)),
                pl.BlockSpec((C, 1), lambda b: (0, 0)),
            ],
            out_specs=pl.BlockSpec((1, C, HW), lambda b: (b, 0, 0)),
            compiler_params=pltpu.CompilerParams(
                dimension_semantics=("parallel",),
                vmem_limit_bytes=56 * 1024 * 1024,
            ),
        )(x3d, w1t, b1t, w2t, b2t)

    return out3d.reshape(B, C, H, W)


# speedup vs baseline: 1.0927x; 1.0927x over previous
import functools

import jax
import jax.numpy as jnp
from jax.experimental import pallas as pl
from jax.experimental.pallas import tpu as pltpu


def _se_block_kernel(x_ref, w1_ref, b1_ref, w2_ref, b2_ref, o_ref, *,
                     nb, inv_hw):
    y = jnp.concatenate(
        [jnp.sum(x_ref[i], axis=1, keepdims=True) for i in range(nb)],
        axis=1,
    ) * inv_hw
    h = jax.lax.dot_general(
        w1_ref[...], y,
        dimension_numbers=(((0,), (0,)), ((), ())),
        preferred_element_type=jnp.float32,
    ) + b1_ref[...].T
    h = jnp.maximum(h, 0.0)
    z = jax.lax.dot_general(
        w2_ref[...], h,
        dimension_numbers=(((0,), (0,)), ((), ())),
        preferred_element_type=jnp.float32,
    ) + b2_ref[...].T
    s = jax.nn.sigmoid(z)
    for i in range(nb):
        o_ref[i] = x_ref[i] * s[:, i:i + 1]


def kernel(x, w1, b1, w2, b2):
    B, C, H, W = x.shape
    HW = H * W
    hidden = w1.shape[1]
    x3d = x.reshape(B, C, HW)

    slab_bytes = C * HW * x.dtype.itemsize
    nb = 1
    for cand in (4, 2):
        if B % cand == 0 and 4 * cand * slab_bytes <= 52 * 1024 * 1024:
            nb = cand
            break

    out3d = pl.pallas_call(
        functools.partial(_se_block_kernel, nb=nb, inv_hw=1.0 / HW),
        out_shape=jax.ShapeDtypeStruct((B, C, HW), x3d.dtype),
        grid=(B // nb,),
        in_specs=[
            pl.BlockSpec((nb, C, HW), lambda b: (b, 0, 0)),
            pl.BlockSpec((C, hidden), lambda b: (0, 0)),
            pl.BlockSpec((1, hidden), lambda b: (0, 0)),
            pl.BlockSpec((hidden, C), lambda b: (0, 0)),
            pl.BlockSpec((1, C), lambda b: (0, 0)),
        ],
        out_specs=pl.BlockSpec((nb, C, HW), lambda b: (b, 0, 0)),
        compiler_params=pltpu.CompilerParams(
            dimension_semantics=("parallel",),
            vmem_limit_bytes=56 * 1024 * 1024,
        ),
    )(x3d, w1, b1, w2, b2)

    return out3d.reshape(B, C, H, W)
